# baseline (speedup 1.0000x reference)
import functools

import jax
import jax.numpy as jnp
from jax import lax
from jax.experimental import pallas as pl
from jax.experimental.pallas import tpu as pltpu
from jax.experimental.pallas import tpu_sc as plsc

_N = 1024
_LOWER = -4.0
_UPPER = 4.0
_DT = 0.01
_B = 131072

_NC = 2
_NS = 16
_NW = _NC * _NS
_L = 16
_ROWS = _B // _NW
_STEPS = _ROWS // _L

_SCALE = _N / (_UPPER - _LOWER)
_BIAS = 0.5 - _LOWER * _SCALE


@functools.partial(
    pl.kernel,
    out_type=jax.ShapeDtypeStruct((_B, 2), jnp.float32),
    mesh=plsc.VectorSubcoreMesh(core_axis_name="c", subcore_axis_name="s"),
    scratch_types=[
        pltpu.VMEM((_ROWS, 2), jnp.float32),
        pltpu.VMEM((_N,), jnp.float32),
    ],
    compiler_params=pltpu.CompilerParams(
        needs_layout_passes=False, use_tc_tiling_on_sc=False
    ),
)
def _sc_kernel(x_hbm, force_hbm, out_hbm, buf, force_v):
    wid = lax.axis_index("s") * _NC + lax.axis_index("c")
    base = wid * _ROWS
    pltpu.sync_copy(force_hbm, force_v)
    pltpu.sync_copy(x_hbm.at[pl.ds(base, _ROWS)], buf)

    lanes = lax.iota(jnp.int32, _L)
    zeros = lanes * 0
    ones = zeros + 1

    def step(j, carry):
        rows = lanes + j * _L
        xv = plsc.load_gather(buf, [rows, zeros])
        u = xv * _SCALE + _BIAS
        u = jnp.minimum(jnp.maximum(u, 0.5), float(_N - 1) + 0.5)
        idx = u.astype(jnp.int32)
        f = plsc.load_gather(force_v, [idx])
        vv = plsc.load_gather(buf, [rows, ones])
        plsc.store_scatter(buf, [rows, ones], vv + f * _DT)
        return carry

    lax.fori_loop(0, _STEPS, step, 0, unroll=8)
    pltpu.sync_copy(buf, out_hbm.at[pl.ds(base, _ROWS)])


def kernel(X, force):
    return _sc_kernel(X, force)


# speedup vs baseline: 1.0817x; 1.0817x over previous
import functools

import jax
import jax.numpy as jnp
from jax import lax
from jax.experimental import pallas as pl
from jax.experimental.pallas import tpu as pltpu
from jax.experimental.pallas import tpu_sc as plsc

_N = 1024
_LOWER = -4.0
_UPPER = 4.0
_DT = 0.01
_B = 131072

_NC = 2
_NS = 16
_NW = _NC * _NS
_L = 16
_ROWS = _B // _NW
_CHUNK = 2 * _ROWS
_STEPS = _ROWS // _L

_SCALE = _N / (_UPPER - _LOWER)
_BIAS = 0.5 - _LOWER * _SCALE


@functools.partial(
    pl.kernel,
    out_type=jax.ShapeDtypeStruct((2 * _B,), jnp.float32),
    mesh=plsc.VectorSubcoreMesh(core_axis_name="c", subcore_axis_name="s"),
    scratch_types=[
        pltpu.VMEM((_CHUNK,), jnp.float32),
        pltpu.VMEM((_N,), jnp.float32),
    ],
    compiler_params=pltpu.CompilerParams(needs_layout_passes=False),
)
def _sc_kernel(x_hbm, force_hbm, out_hbm, buf, force_v):
    wid = lax.axis_index("s") * _NC + lax.axis_index("c")
    base = wid * _CHUNK
    pltpu.sync_copy(force_hbm, force_v)
    pltpu.sync_copy(x_hbm.at[pl.ds(base, _CHUNK)], buf)

    evens = lax.iota(jnp.int32, _L) * 2

    def step(j, carry):
        xi = evens + j * (2 * _L)
        xv = plsc.load_gather(buf, [xi])
        u = xv * _SCALE + _BIAS
        u = jnp.minimum(jnp.maximum(u, 0.5), float(_N - 1) + 0.5)
        idx = u.astype(jnp.int32)
        f = plsc.load_gather(force_v, [idx])
        vi = xi + 1
        vv = plsc.load_gather(buf, [vi])
        plsc.store_scatter(buf, [vi], vv + f * _DT)
        return carry

    lax.fori_loop(0, _STEPS, step, 0, unroll=8)
    pltpu.sync_copy(buf, out_hbm.at[pl.ds(base, _CHUNK)])


def kernel(X, force):
    return _sc_kernel(X.reshape(2 * _B), force).reshape(_B, 2)


# speedup vs baseline: 1.0825x; 1.0007x over previous
import functools

import jax
import jax.numpy as jnp
from jax import lax
from jax.experimental import pallas as pl
from jax.experimental.pallas import tpu as pltpu
from jax.experimental.pallas import tpu_sc as plsc

_N = 1024
_LOWER = -4.0
_UPPER = 4.0
_DT = 0.01
_B = 131072

_NC = 2
_NS = 16
_NW = _NC * _NS
_L = 16
_ROWS = _B // _NW
_CHUNK = 2 * _ROWS
_STEPS = _ROWS // _L

_SCALE = _N / (_UPPER - _LOWER)
_BIAS = 0.5 - _LOWER * _SCALE


@functools.partial(
    pl.kernel,
    out_type=jax.ShapeDtypeStruct((2 * _B,), jnp.float32),
    mesh=plsc.VectorSubcoreMesh(core_axis_name="c", subcore_axis_name="s"),
    scratch_types=[
        pltpu.VMEM((_CHUNK,), jnp.float32),
        pltpu.VMEM((_N,), jnp.float32),
    ],
    compiler_params=pltpu.CompilerParams(
        needs_layout_passes=False, use_tc_tiling_on_sc=False
    ),
)
def _sc_kernel(x_hbm, force_hbm, out_hbm, buf, force_v):
    wid = lax.axis_index("s") * _NC + lax.axis_index("c")
    base = wid * _CHUNK
    pltpu.sync_copy(force_hbm, force_v)
    pltpu.sync_copy(x_hbm.at[pl.ds(base, _CHUNK)], buf)

    evens = lax.iota(jnp.int32, _L) * 2

    def step(j, carry):
        xi = evens + j * (2 * _L)
        xv = plsc.load_gather(buf, [xi])
        u = xv * _SCALE + _BIAS
        u = jnp.minimum(jnp.maximum(u, 0.5), float(_N - 1) + 0.5)
        idx = u.astype(jnp.int32)
        f = plsc.load_gather(force_v, [idx])
        vi = xi + 1
        vv = plsc.load_gather(buf, [vi])
        plsc.store_scatter(buf, [vi], vv + f * _DT)
        return carry

    lax.fori_loop(0, _STEPS, step, 0, unroll=8)
    pltpu.sync_copy(buf, out_hbm.at[pl.ds(base, _CHUNK)])


def kernel(X, force):
    return _sc_kernel(X.reshape(2 * _B), force).reshape(_B, 2)


# speedup vs baseline: 7.1785x; 6.6316x over previous
import functools

import jax
import jax.numpy as jnp
from jax import lax
from jax.experimental import pallas as pl
from jax.experimental.pallas import tpu as pltpu
from jax.experimental.pallas import tpu_sc as plsc

_N = 1024
_LOWER = -4.0
_UPPER = 4.0
_DT = 0.01
_B = 131072

_NC = 2
_NS = 16
_NW = _NC * _NS
_L = 16
_ROWS = _B // _NW
_STEPS = _ROWS // _L

_SCALE = _N / (_UPPER - _LOWER)
_BIAS = 0.5 - _LOWER * _SCALE


@functools.partial(
    pl.kernel,
    out_type=jax.ShapeDtypeStruct((_B,), jnp.float32),
    mesh=plsc.VectorSubcoreMesh(core_axis_name="c", subcore_axis_name="s"),
    scratch_types=[
        pltpu.VMEM((_ROWS,), jnp.float32),
        pltpu.VMEM((_ROWS,), jnp.float32),
        pltpu.VMEM((_N,), jnp.float32),
    ],
    compiler_params=pltpu.CompilerParams(needs_layout_passes=False),
)
def _sc_kernel(x_hbm, v_hbm, force_hbm, out_hbm, xbuf, vbuf, force_v):
    wid = lax.axis_index("s") * _NC + lax.axis_index("c")
    base = wid * _ROWS
    pltpu.sync_copy(x_hbm.at[pl.ds(base, _ROWS)], xbuf)
    pltpu.sync_copy(v_hbm.at[pl.ds(base, _ROWS)], vbuf)
    pltpu.sync_copy(force_hbm, force_v)

    def step(j, carry):
        sl = pl.ds(j * _L, _L)
        u = xbuf[sl] * _SCALE + _BIAS
        u = jnp.minimum(jnp.maximum(u, 0.5), float(_N - 1) + 0.5)
        idx = u.astype(jnp.int32)
        f = plsc.load_gather(force_v, [idx])
        vbuf[sl] = vbuf[sl] + f * _DT
        return carry

    lax.fori_loop(0, _STEPS, step, 0, unroll=8)
    pltpu.sync_copy(vbuf, out_hbm.at[pl.ds(base, _ROWS)])


def kernel(X, force):
    x = X[:, 0]
    v_new = _sc_kernel(x, X[:, 1], force)
    return jnp.stack([x, v_new], axis=1)


# speedup vs baseline: 9.2996x; 1.2955x over previous
import functools

import jax
import jax.numpy as jnp
from jax import lax
from jax.experimental import pallas as pl
from jax.experimental.pallas import tpu as pltpu
from jax.experimental.pallas import tpu_sc as plsc

_N = 1024
_LOWER = -4.0
_UPPER = 4.0
_DT = 0.01
_B = 131072

_NC = 2
_NS = 16
_NW = _NC * _NS
_L = 16
_NB = _B // 128
_BLOCKS = _NB // _NW
_VPB = 128 // _L

_SCALE = _N / (_UPPER - _LOWER)
_BIAS = 0.5 - _LOWER * _SCALE


@functools.partial(
    pl.kernel,
    out_type=jax.ShapeDtypeStruct((_NB, 2, 128), jnp.float32),
    mesh=plsc.VectorSubcoreMesh(core_axis_name="c", subcore_axis_name="s"),
    scratch_types=[
        pltpu.VMEM((_BLOCKS, 2, 128), jnp.float32),
        pltpu.VMEM((_N,), jnp.float32),
    ],
    compiler_params=pltpu.CompilerParams(
        needs_layout_passes=False, use_tc_tiling_on_sc=False
    ),
)
def _sc_kernel(x_hbm, force_hbm, out_hbm, buf, force_v):
    wid = lax.axis_index("s") * _NC + lax.axis_index("c")
    base = wid * _BLOCKS
    pltpu.sync_copy(x_hbm.at[pl.ds(base, _BLOCKS)], buf)
    pltpu.sync_copy(force_hbm, force_v)

    def step(j, carry):
        for k in range(_VPB):
            sl = pl.ds(k * _L, _L)
            u = buf[j, 0, sl] * _SCALE + _BIAS
            u = jnp.minimum(jnp.maximum(u, 0.5), float(_N - 1) + 0.5)
            idx = u.astype(jnp.int32)
            f = plsc.load_gather(force_v, [idx])
            buf[j, 1, sl] = buf[j, 1, sl] + f * _DT
        return carry

    lax.fori_loop(0, _BLOCKS, step, 0)
    pltpu.sync_copy(buf, out_hbm.at[pl.ds(base, _BLOCKS)])


def kernel(X, force):
    xb = X.reshape(_NB, 128, 2).transpose(0, 2, 1)
    ob = _sc_kernel(xb, force)
    return ob.transpose(0, 2, 1).reshape(_B, 2)


# speedup vs baseline: 10.0721x; 1.0831x over previous
import functools

import jax
import jax.numpy as jnp
from jax import lax
from jax.experimental import pallas as pl
from jax.experimental.pallas import tpu as pltpu
from jax.experimental.pallas import tpu_sc as plsc

_N = 1024
_LOWER = -4.0
_UPPER = 4.0
_DT = 0.01
_B = 131072

_NC = 2
_NS = 16
_NW = _NC * _NS
_L = 16
_NB = _B // 128
_BLOCKS = _NB // _NW
_VPB = 128 // _L

_SCALE = _N / (_UPPER - _LOWER)
_BIAS = 0.5 - _LOWER * _SCALE


@functools.partial(
    pl.kernel,
    out_type=jax.ShapeDtypeStruct((_NB, 2, 128), jnp.float32),
    mesh=plsc.VectorSubcoreMesh(core_axis_name="c", subcore_axis_name="s"),
    scratch_types=[
        pltpu.VMEM((_BLOCKS, 2, 128), jnp.float32),
        pltpu.VMEM((_N,), jnp.float32),
        pltpu.SemaphoreType.DMA,
    ],
    compiler_params=pltpu.CompilerParams(
        needs_layout_passes=False, use_tc_tiling_on_sc=False
    ),
)
def _sc_kernel(x_hbm, force_hbm, out_hbm, buf, force_v, sem):
    wid = lax.axis_index("s") * _NC + lax.axis_index("c")
    base = wid * _BLOCKS
    slab = pltpu.async_copy(x_hbm.at[pl.ds(base, _BLOCKS)], buf, sem)
    pltpu.sync_copy(force_hbm, force_v)

    @plsc.parallel_loop(0, _N // _L, unroll=8)
    def prescale(i):
        sl = pl.ds(i * _L, _L)
        force_v[sl] = force_v[sl] * _DT

    slab.wait()

    @plsc.parallel_loop(0, _BLOCKS * _VPB, unroll=8)
    def step(i):
        j = i // _VPB
        sl = pl.ds((i % _VPB) * _L, _L)
        u = buf[j, 0, sl] * _SCALE + _BIAS
        u = jnp.minimum(jnp.maximum(u, 0.5), float(_N - 1) + 0.5)
        idx = u.astype(jnp.int32)
        buf[j, 1, sl] = buf[j, 1, sl] + plsc.load_gather(force_v, [idx])

    pltpu.sync_copy(buf, out_hbm.at[pl.ds(base, _BLOCKS)])


def kernel(X, force):
    xb = X.reshape(_NB, 128, 2).transpose(0, 2, 1)
    ob = _sc_kernel(xb, force)
    return ob.transpose(0, 2, 1).reshape(_B, 2)
